# baseline (speedup 1.0000x reference)
import functools

import jax
import jax.numpy as jnp
from jax import lax
from jax.experimental import pallas as pl
from jax.experimental.pallas import tpu as pltpu
from jax.experimental.pallas import tpu_sc as plsc

VOCAB = 1000000
D_MODEL = 32
BATCH = 16384
HIST = 50

N_ROWS = BATCH * HIST
N_ELEMS = N_ROWS * D_MODEL

NUM_CORES = 2
NUM_SUBCORES = 16
NW = NUM_CORES * NUM_SUBCORES
ROWS_PER_W = N_ROWS // NW
IDX_W = 128
K_GATHERS = 20
CHUNK_ROWS = IDX_W * K_GATHERS
N_OUTER = ROWS_PER_W // CHUNK_ROWS

LANES = 128
TC_ROWS = N_ELEMS // LANES
TC_BLOCK = 1024
TC_GRID = TC_ROWS // TC_BLOCK

FULL_BLOCKS = 89
BITS1_BLOCKS = 54
BITS2_BLOCKS = TC_GRID - FULL_BLOCKS - BITS1_BLOCKS
BITS_BLOCKS = BITS1_BLOCKS + BITS2_BLOCKS
E0 = FULL_BLOCKS * TC_BLOCK * LANES
E1 = (FULL_BLOCKS + BITS1_BLOCKS) * TC_BLOCK * LANES
BCHUNK = 4096
B_UNROLL = 8
N_J = BCHUNK // (16 * B_UNROLL)


def _threefry_bits(e):
    ks0 = jnp.uint32(0)
    ks1 = jnp.uint32(42)
    ks2 = jnp.uint32(0x1BD11BDA) ^ ks0 ^ ks1
    ks = (ks0, ks1, ks2)
    rot = ((13, 15, 26, 6), (17, 29, 16, 24))
    x0 = jnp.zeros_like(e) + ks[0]
    x1 = e + ks[1]
    for i in range(5):
        for r in rot[i % 2]:
            x0 = x0 + x1
            x1 = (x1 << r) | (x1 >> (32 - r))
            x1 = x1 ^ x0
        x0 = x0 + ks[(i + 1) % 3]
        x1 = x1 + ks[(i + 2) % 3] + jnp.uint32(i + 1)
    return x0 ^ x1


VPAD = 1 << 20
STRIP = VPAD // 4
T_ROWS = 1024
T_GRID = STRIP // T_ROWS


def _tc_table_transpose(table_t):
    def body(i0_ref, i1_ref, i2_ref, i3_ref, out_ref):
        for s, ref in enumerate((i0_ref, i1_ref, i2_ref, i3_ref)):
            out_ref[:, s * 32:(s + 1) * 32] = ref[...].T

    return pl.pallas_call(
        body,
        grid=(T_GRID,),
        in_specs=[
            pl.BlockSpec((32, T_ROWS),
                         lambda i, s=s: (0, jnp.minimum(s * T_GRID + i,
                                                        (VOCAB - 1) // T_ROWS)))
            for s in range(4)
        ],
        out_specs=pl.BlockSpec((T_ROWS, 128), lambda i: (i, 0)),
        out_shape=jax.ShapeDtypeStruct((STRIP, 128), jnp.float32),
        compiler_params=pltpu.CompilerParams(
            dimension_semantics=("arbitrary",),
        ),
    )(table_t, table_t, table_t, table_t)


def _sc_gather(x_flat, table):
    mesh = plsc.VectorSubcoreMesh(core_axis_name="c", subcore_axis_name="s")

    @functools.partial(
        pl.kernel,
        mesh=mesh,
        out_type=jax.ShapeDtypeStruct((N_ROWS, D_MODEL), jnp.float32),
        scratch_types=[
            pltpu.VMEM((CHUNK_ROWS,), jnp.int32),
            pltpu.VMEM((CHUNK_ROWS, D_MODEL), jnp.float32),
            pltpu.SemaphoreType.DMA,
        ],
        compiler_params=pltpu.CompilerParams(use_tc_tiling_on_sc=False),
    )
    def k(x_hbm, table_hbm, out_hbm, idx_v, rows_v, sem):
        wid = lax.axis_index("s") * NUM_CORES + lax.axis_index("c")

        def step(g, _):
            row0 = wid * ROWS_PER_W + g * CHUNK_ROWS
            pltpu.sync_copy(x_hbm.at[pl.ds(row0, CHUNK_ROWS)], idx_v)

            def fix(jv, _2):
                v = idx_v[pl.ds(jv * 16, 16)]
                idx_v[pl.ds(jv * 16, 16)] = (
                    ((v & jnp.int32(STRIP - 1)) << 2) | (v >> 18))
                return _2

            lax.fori_loop(0, CHUNK_ROWS // 16, fix, None)
            copies = [
                pltpu.async_copy(
                    table_hbm.at[idx_v.at[pl.ds(j * IDX_W, IDX_W)]],
                    rows_v.at[pl.ds(j * IDX_W, IDX_W)],
                    sem,
                )
                for j in range(K_GATHERS)
            ]
            for c in copies:
                c.wait()
            pltpu.sync_copy(rows_v, out_hbm.at[pl.ds(row0, CHUNK_ROWS)])
            return _

        lax.fori_loop(0, N_OUTER, step, None)

    return k(x_flat, table)


def _sc_bits(e_start, n_blocks, dep=None):
    n_elems = n_blocks * TC_BLOCK * LANES
    per_w = n_elems // NW
    n_chunk = per_w // BCHUNK
    mesh = plsc.VectorSubcoreMesh(core_axis_name="c", subcore_axis_name="s")

    def body(*refs):
        out_hbm, buf = refs[-2], refs[-1]
        wid = lax.axis_index("s") * NUM_CORES + lax.axis_index("c")
        base_w = wid * per_w
        lane = lax.iota(jnp.int32, 16).astype(jnp.uint32)

        def chunk(c, _):
            def inner(j, _2):
                off = j * (16 * B_UNROLL)
                e_base = jnp.uint32(e_start) + jnp.uint32(base_w) + (
                    c.astype(jnp.uint32) * jnp.uint32(BCHUNK)
                    + off.astype(jnp.uint32))
                for u_ in range(B_UNROLL):
                    e = e_base + jnp.uint32(u_ * 16) + lane
                    buf[pl.ds(off + u_ * 16, 16)] = _threefry_bits(e)
                return _2

            lax.fori_loop(0, N_J, inner, None)
            pltpu.sync_copy(buf, out_hbm.at[pl.ds(base_w + c * BCHUNK, BCHUNK)])
            return _

        lax.fori_loop(0, n_chunk, chunk, None)

    k = functools.partial(
        pl.kernel,
        mesh=mesh,
        out_type=jax.ShapeDtypeStruct((n_elems,), jnp.uint32),
        scratch_types=[
            pltpu.VMEM((BCHUNK,), jnp.uint32),
        ],
        compiler_params=pltpu.CompilerParams(use_tc_tiling_on_sc=False),
    )(body)
    return k(dep) if dep is not None else k()


_U_LO = -0.9999999403953552
_SQRT2 = 1.4142135623730951

SUB = 64
UNROLL = 2
STEP_ROWS = SUB * UNROLL
N_SUB = TC_BLOCK // STEP_ROWS


def _scales(ease_ref):
    avg = jnp.sum(ease_ref[0, :]) * jnp.float32(0.125)
    s_noise = jnp.float32(0.2) * (jnp.float32(1.0) - avg)
    s_out = jnp.float32(0.5) + avg
    return avg, s_noise, s_out


def _bits_to_out(bits, emb, s_noise, s_out):
    fb = (bits >> 9) | jnp.uint32(0x3F800000)
    f = lax.bitcast_convert_type(fb, jnp.float32) - jnp.float32(1.0)
    u = jnp.maximum(jnp.float32(_U_LO),
                    f * jnp.float32(1.0 - _U_LO) + jnp.float32(_U_LO))
    noise = jnp.float32(_SQRT2) * lax.erf_inv(u)
    return (emb + noise * s_noise) * s_out


def _tc_body_full(ease_ref, emb_ref, out_ref, avg_ref):
    avg, s_noise, s_out = _scales(ease_ref)
    i = pl.program_id(0)
    r = lax.broadcasted_iota(jnp.int32, (SUB, LANES), 0)
    c = lax.broadcasted_iota(jnp.int32, (SUB, LANES), 1)
    lin = (r << 7) + c

    def sub(k, _):
        for u_ in range(UNROLL):
            row0 = k * STEP_ROWS + u_ * SUB
            e0 = (i * TC_BLOCK + row0) << 7
            bits = _threefry_bits((e0 + lin).astype(jnp.uint32))
            out_ref[pl.ds(row0, SUB), :] = _bits_to_out(
                bits, emb_ref[pl.ds(row0, SUB), :], s_noise, s_out)
        return _

    lax.fori_loop(0, N_SUB, sub, None)
    avg_ref[...] = jnp.reshape(avg, (1, 1))


def _tc_body_bits(ease_ref, _aliased_ref, emb_ref, bits1_ref, bits2_ref,
                  out_ref):
    _, s_noise, s_out = _scales(ease_ref)
    i = pl.program_id(0)

    def sub(k, _):
        for u_ in range(UNROLL):
            row0 = k * STEP_ROWS + u_ * SUB
            bits = lax.cond(
                i < BITS1_BLOCKS,
                lambda: bits1_ref[pl.ds(row0, SUB), :],
                lambda: bits2_ref[pl.ds(row0, SUB), :])
            out_ref[pl.ds(row0, SUB), :] = _bits_to_out(
                bits, emb_ref[pl.ds(row0, SUB), :], s_noise, s_out)
        return _

    lax.fori_loop(0, N_SUB, sub, None)


def _tc_combine(ease2, emb2d, bits1_2d, bits2_2d):
    out_a, avg = pl.pallas_call(
        _tc_body_full,
        grid=(FULL_BLOCKS,),
        in_specs=[
            pl.BlockSpec((1, 8), lambda i: (0, 0)),
            pl.BlockSpec((TC_BLOCK, LANES), lambda i: (i, 0)),
        ],
        out_specs=[
            pl.BlockSpec((TC_BLOCK, LANES), lambda i: (i, 0)),
            pl.BlockSpec((1, 1), lambda i: (0, 0)),
        ],
        out_shape=[
            jax.ShapeDtypeStruct((TC_ROWS, LANES), jnp.float32),
            jax.ShapeDtypeStruct((1, 1), jnp.float32),
        ],
        compiler_params=pltpu.CompilerParams(
            dimension_semantics=("arbitrary",),
        ),
    )(ease2, emb2d)

    out = pl.pallas_call(
        _tc_body_bits,
        grid=(BITS_BLOCKS,),
        in_specs=[
            pl.BlockSpec((1, 8), lambda i: (0, 0)),
            pl.BlockSpec(memory_space=pl.ANY),
            pl.BlockSpec((TC_BLOCK, LANES), lambda i: (i + FULL_BLOCKS, 0)),
            pl.BlockSpec((TC_BLOCK, LANES),
                         lambda i: (jnp.minimum(i, BITS1_BLOCKS - 1), 0)),
            pl.BlockSpec((TC_BLOCK, LANES),
                         lambda i: (jnp.maximum(i - BITS1_BLOCKS, 0), 0)),
        ],
        out_specs=[
            pl.BlockSpec((TC_BLOCK, LANES), lambda i: (i + FULL_BLOCKS, 0)),
        ],
        out_shape=[
            jax.ShapeDtypeStruct((TC_ROWS, LANES), jnp.float32),
        ],
        input_output_aliases={1: 0},
        compiler_params=pltpu.CompilerParams(
            dimension_semantics=("arbitrary",),
        ),
    )(ease2, out_a, emb2d, bits1_2d, bits2_2d)
    return out[0], avg


def kernel(x, ease_scores, table):
    x_flat = x.astype(jnp.int32).reshape(N_ROWS)
    t128 = _tc_table_transpose(lax.transpose(table, (1, 0)))
    emb = _sc_gather(x_flat, t128.reshape(VPAD, D_MODEL))
    bits1 = _sc_bits(E0, BITS1_BLOCKS)
    bits2 = _sc_bits(E1, BITS2_BLOCKS, dep=emb)
    emb2d = emb.reshape(TC_ROWS, LANES)
    bits1_2d = bits1.reshape(BITS1_BLOCKS * TC_BLOCK, LANES)
    bits2_2d = bits2.reshape(BITS2_BLOCKS * TC_BLOCK, LANES)
    out2d, avg = _tc_combine(ease_scores.reshape(1, 8), emb2d,
                             bits1_2d, bits2_2d)
    return out2d.reshape(BATCH, HIST, D_MODEL), avg.reshape(())


# speedup vs baseline: 1.2212x; 1.2212x over previous
import functools

import jax
import jax.numpy as jnp
from jax import lax
from jax.experimental import pallas as pl
from jax.experimental.pallas import tpu as pltpu
from jax.experimental.pallas import tpu_sc as plsc

VOCAB = 1000000
D_MODEL = 32
BATCH = 16384
HIST = 50

N_ROWS = BATCH * HIST
N_ELEMS = N_ROWS * D_MODEL

NUM_CORES = 2
NUM_SUBCORES = 16
NW = NUM_CORES * NUM_SUBCORES
ROWS_PER_W = N_ROWS // NW
IDX_W = 128
K_GATHERS = 20
CHUNK_ROWS = IDX_W * K_GATHERS
N_OUTER = ROWS_PER_W // CHUNK_ROWS

LANES = 128
TC_ROWS = N_ELEMS // LANES
TC_BLOCK = 1024
TC_GRID = TC_ROWS // TC_BLOCK

FULL_BLOCKS = 89
BITS1_BLOCKS = 54
BITS2_BLOCKS = TC_GRID - FULL_BLOCKS - BITS1_BLOCKS
BITS_BLOCKS = BITS1_BLOCKS + BITS2_BLOCKS
E0 = FULL_BLOCKS * TC_BLOCK * LANES
E1 = (FULL_BLOCKS + BITS1_BLOCKS) * TC_BLOCK * LANES
BCHUNK = 4096
B_UNROLL = 8
N_J = BCHUNK // (16 * B_UNROLL)


def _threefry_bits(e):
    ks0 = jnp.uint32(0)
    ks1 = jnp.uint32(42)
    ks2 = jnp.uint32(0x1BD11BDA) ^ ks0 ^ ks1
    ks = (ks0, ks1, ks2)
    rot = ((13, 15, 26, 6), (17, 29, 16, 24))
    x0 = jnp.zeros_like(e) + ks[0]
    x1 = e + ks[1]
    for i in range(5):
        for r in rot[i % 2]:
            x0 = x0 + x1
            x1 = (x1 << r) | (x1 >> (32 - r))
            x1 = x1 ^ x0
        x0 = x0 + ks[(i + 1) % 3]
        x1 = x1 + ks[(i + 2) % 3] + jnp.uint32(i + 1)
    return x0 ^ x1


VPAD = 1 << 20
STRIP = VPAD // 4
T_ROWS = 1024
T_GRID = STRIP // T_ROWS


def _tc_table_transpose(table_t):
    def body(i0_ref, i1_ref, i2_ref, i3_ref, out_ref):
        for s, ref in enumerate((i0_ref, i1_ref, i2_ref, i3_ref)):
            out_ref[:, s * 32:(s + 1) * 32] = ref[...].T

    return pl.pallas_call(
        body,
        grid=(T_GRID,),
        in_specs=[
            pl.BlockSpec((32, T_ROWS),
                         lambda i, s=s: (0, jnp.minimum(s * T_GRID + i,
                                                        (VOCAB - 1) // T_ROWS)))
            for s in range(4)
        ],
        out_specs=pl.BlockSpec((T_ROWS, 128), lambda i: (i, 0)),
        out_shape=jax.ShapeDtypeStruct((STRIP, 128), jnp.float32),
        compiler_params=pltpu.CompilerParams(
            dimension_semantics=("arbitrary",),
        ),
    )(table_t, table_t, table_t, table_t)


def _sc_gather(x_flat, table, dep):
    mesh = plsc.VectorSubcoreMesh(core_axis_name="c", subcore_axis_name="s")

    @functools.partial(
        pl.kernel,
        mesh=mesh,
        out_type=jax.ShapeDtypeStruct((N_ROWS, D_MODEL), jnp.float32),
        scratch_types=[
            pltpu.VMEM((CHUNK_ROWS,), jnp.int32),
            pltpu.VMEM((CHUNK_ROWS, D_MODEL), jnp.float32),
            pltpu.SemaphoreType.DMA,
        ],
        compiler_params=pltpu.CompilerParams(use_tc_tiling_on_sc=False),
    )
    def k(x_hbm, table_hbm, dep_hbm, out_hbm, idx_v, rows_v, sem):
        wid = lax.axis_index("s") * NUM_CORES + lax.axis_index("c")

        def step(g, _):
            row0 = wid * ROWS_PER_W + g * CHUNK_ROWS
            pltpu.sync_copy(x_hbm.at[pl.ds(row0, CHUNK_ROWS)], idx_v)

            def fix(jv, _2):
                v = idx_v[pl.ds(jv * 16, 16)]
                idx_v[pl.ds(jv * 16, 16)] = (
                    ((v & jnp.int32(STRIP - 1)) << 2) | (v >> 18))
                return _2

            lax.fori_loop(0, CHUNK_ROWS // 16, fix, None)
            copies = [
                pltpu.async_copy(
                    table_hbm.at[idx_v.at[pl.ds(j * IDX_W, IDX_W)]],
                    rows_v.at[pl.ds(j * IDX_W, IDX_W)],
                    sem,
                )
                for j in range(K_GATHERS)
            ]
            for c in copies:
                c.wait()
            pltpu.sync_copy(rows_v, out_hbm.at[pl.ds(row0, CHUNK_ROWS)])
            return _

        lax.fori_loop(0, N_OUTER, step, None)

    return k(x_flat, table, dep)


def _sc_bits(e_start, n_blocks, dep=None):
    n_elems = n_blocks * TC_BLOCK * LANES
    per_w = n_elems // NW
    n_chunk = per_w // BCHUNK
    mesh = plsc.VectorSubcoreMesh(core_axis_name="c", subcore_axis_name="s")

    def body(*refs):
        out_hbm, buf = refs[-2], refs[-1]
        wid = lax.axis_index("s") * NUM_CORES + lax.axis_index("c")
        base_w = wid * per_w
        lane = lax.iota(jnp.int32, 16).astype(jnp.uint32)

        def chunk(c, _):
            def inner(j, _2):
                off = j * (16 * B_UNROLL)
                e_base = jnp.uint32(e_start) + jnp.uint32(base_w) + (
                    c.astype(jnp.uint32) * jnp.uint32(BCHUNK)
                    + off.astype(jnp.uint32))
                for u_ in range(B_UNROLL):
                    e = e_base + jnp.uint32(u_ * 16) + lane
                    buf[pl.ds(off + u_ * 16, 16)] = _threefry_bits(e)
                return _2

            lax.fori_loop(0, N_J, inner, None)
            pltpu.sync_copy(buf, out_hbm.at[pl.ds(base_w + c * BCHUNK, BCHUNK)])
            return _

        lax.fori_loop(0, n_chunk, chunk, None)

    k = functools.partial(
        pl.kernel,
        mesh=mesh,
        out_type=jax.ShapeDtypeStruct((n_elems,), jnp.uint32),
        scratch_types=[
            pltpu.VMEM((BCHUNK,), jnp.uint32),
        ],
        compiler_params=pltpu.CompilerParams(use_tc_tiling_on_sc=False),
    )(body)
    return k(dep) if dep is not None else k()


_U_LO = -0.9999999403953552
_SQRT2 = 1.4142135623730951

SUB = 64
UNROLL = 2
STEP_ROWS = SUB * UNROLL
N_SUB = TC_BLOCK // STEP_ROWS


def _scales(ease_ref):
    avg = jnp.sum(ease_ref[0, :]) * jnp.float32(0.125)
    s_noise = jnp.float32(0.2) * (jnp.float32(1.0) - avg)
    s_out = jnp.float32(0.5) + avg
    return avg, s_noise, s_out


def _bits_to_out(bits, emb, s_noise, s_out):
    fb = (bits >> 9) | jnp.uint32(0x3F800000)
    f = lax.bitcast_convert_type(fb, jnp.float32) - jnp.float32(1.0)
    u = jnp.maximum(jnp.float32(_U_LO),
                    f * jnp.float32(1.0 - _U_LO) + jnp.float32(_U_LO))
    noise = jnp.float32(_SQRT2) * lax.erf_inv(u)
    return (emb + noise * s_noise) * s_out


def _tc_body_full(ease_ref, emb_ref, out_ref, avg_ref):
    avg, s_noise, s_out = _scales(ease_ref)
    i = pl.program_id(0)
    r = lax.broadcasted_iota(jnp.int32, (SUB, LANES), 0)
    c = lax.broadcasted_iota(jnp.int32, (SUB, LANES), 1)
    lin = (r << 7) + c

    def sub(k, _):
        for u_ in range(UNROLL):
            row0 = k * STEP_ROWS + u_ * SUB
            e0 = (i * TC_BLOCK + row0) << 7
            bits = _threefry_bits((e0 + lin).astype(jnp.uint32))
            out_ref[pl.ds(row0, SUB), :] = _bits_to_out(
                bits, emb_ref[pl.ds(row0, SUB), :], s_noise, s_out)
        return _

    lax.fori_loop(0, N_SUB, sub, None)
    avg_ref[...] = jnp.reshape(avg, (1, 1))


def _tc_body_bits(ease_ref, _aliased_ref, emb_ref, bits1_ref, bits2_ref,
                  out_ref):
    _, s_noise, s_out = _scales(ease_ref)
    i = pl.program_id(0)

    def sub(k, _):
        for u_ in range(UNROLL):
            row0 = k * STEP_ROWS + u_ * SUB
            bits = lax.cond(
                i < BITS1_BLOCKS,
                lambda: bits1_ref[pl.ds(row0, SUB), :],
                lambda: bits2_ref[pl.ds(row0, SUB), :])
            out_ref[pl.ds(row0, SUB), :] = _bits_to_out(
                bits, emb_ref[pl.ds(row0, SUB), :], s_noise, s_out)
        return _

    lax.fori_loop(0, N_SUB, sub, None)


def _tc_combine(ease2, emb2d, bits1_2d, bits2_2d):
    out_a, avg = pl.pallas_call(
        _tc_body_full,
        grid=(FULL_BLOCKS,),
        in_specs=[
            pl.BlockSpec((1, 8), lambda i: (0, 0)),
            pl.BlockSpec((TC_BLOCK, LANES), lambda i: (i, 0)),
        ],
        out_specs=[
            pl.BlockSpec((TC_BLOCK, LANES), lambda i: (i, 0)),
            pl.BlockSpec((1, 1), lambda i: (0, 0)),
        ],
        out_shape=[
            jax.ShapeDtypeStruct((TC_ROWS, LANES), jnp.float32),
            jax.ShapeDtypeStruct((1, 1), jnp.float32),
        ],
        compiler_params=pltpu.CompilerParams(
            dimension_semantics=("arbitrary",),
        ),
    )(ease2, emb2d)

    out = pl.pallas_call(
        _tc_body_bits,
        grid=(BITS_BLOCKS,),
        in_specs=[
            pl.BlockSpec((1, 8), lambda i: (0, 0)),
            pl.BlockSpec(memory_space=pl.ANY),
            pl.BlockSpec((TC_BLOCK, LANES), lambda i: (i + FULL_BLOCKS, 0)),
            pl.BlockSpec((TC_BLOCK, LANES),
                         lambda i: (jnp.minimum(i, BITS1_BLOCKS - 1), 0)),
            pl.BlockSpec((TC_BLOCK, LANES),
                         lambda i: (jnp.maximum(i - BITS1_BLOCKS, 0), 0)),
        ],
        out_specs=[
            pl.BlockSpec((TC_BLOCK, LANES), lambda i: (i + FULL_BLOCKS, 0)),
        ],
        out_shape=[
            jax.ShapeDtypeStruct((TC_ROWS, LANES), jnp.float32),
        ],
        input_output_aliases={1: 0},
        compiler_params=pltpu.CompilerParams(
            dimension_semantics=("arbitrary",),
        ),
    )(ease2, out_a, emb2d, bits1_2d, bits2_2d)
    return out[0], avg


def kernel(x, ease_scores, table):
    x_flat = x.astype(jnp.int32).reshape(N_ROWS)
    t128 = _tc_table_transpose(lax.transpose(table, (1, 0)))
    bits1 = _sc_bits(E0, BITS1_BLOCKS)
    emb = _sc_gather(x_flat, t128.reshape(VPAD, D_MODEL), bits1)
    bits2 = _sc_bits(E1, BITS2_BLOCKS, dep=emb)
    emb2d = emb.reshape(TC_ROWS, LANES)
    bits1_2d = bits1.reshape(BITS1_BLOCKS * TC_BLOCK, LANES)
    bits2_2d = bits2.reshape(BITS2_BLOCKS * TC_BLOCK, LANES)
    out2d, avg = _tc_combine(ease_scores.reshape(1, 8), emb2d,
                             bits1_2d, bits2_2d)
    return out2d.reshape(BATCH, HIST, D_MODEL), avg.reshape(())
